# baseline (speedup 1.0000x reference)
import functools

import jax
import jax.numpy as jnp
from jax.experimental import pallas as pl
from jax.experimental.pallas import tpu as pltpu

N = 10000
E = 160000
F = 128
HID = 256
LAT = 128
OUT = 128
HEADS = 2

ROW_BLK = 1000


def _mm_kernel(x_ref, w_ref, o_ref):
    o_ref[...] = jnp.dot(x_ref[...], w_ref[...],
                         preferred_element_type=jnp.float32)


def _matmul(x, w):
    n, k = x.shape
    k2, m = w.shape
    grid = n // ROW_BLK
    return pl.pallas_call(
        _mm_kernel,
        grid=(grid,),
        in_specs=[
            pl.BlockSpec((ROW_BLK, k), lambda i: (i, 0)),
            pl.BlockSpec((k, m), lambda i: (0, 0)),
        ],
        out_specs=pl.BlockSpec((ROW_BLK, m), lambda i: (i, 0)),
        out_shape=jax.ShapeDtypeStruct((n, m), jnp.float32),
    )(x, w)


def _feast(x, src, dst, W, u, c, b, heads, out_ch):
    u_pad = jnp.zeros((u.shape[0], 128), jnp.float32).at[:, :heads].set(u)
    xu = _matmul(x, u_pad)[:, :heads]
    xw = _matmul(x, W)
    q = jax.nn.softmax(xu[src] - xu[dst] + c, axis=-1)
    xw_s = xw[src].reshape(src.shape[0], heads, out_ch)
    msg = jnp.sum(xw_s * q[:, :, None], axis=1)
    s = jax.ops.segment_sum(msg, dst, num_segments=N)
    cnt = jax.ops.segment_sum(jnp.ones((dst.shape[0],), jnp.float32), dst,
                              num_segments=N)
    return s / jnp.clip(cnt, 1.0, None)[:, None] + b


def _gcn(x, src, dst, norm, W, b):
    xw = _matmul(x, W)
    out = jax.ops.segment_sum(norm[:, None] * xw[src], dst, num_segments=N)
    return out + b


def kernel(x, edge_index, edge_weight, W1, u1, c1, b1, W2, u2, c2, b2, Wl, bl,
           We1, be1, Wmu, bmu, Wlv, blv, Wd1, bd1, Wd2, bd2):
    loop = jnp.arange(N, dtype=edge_index.dtype)
    src = jnp.concatenate([edge_index[0], loop])
    dst = jnp.concatenate([edge_index[1], loop])
    ew = jnp.concatenate([edge_weight, jnp.ones((N,), jnp.float32)])

    h = jax.nn.relu(_feast(x, src, dst, W1, u1, c1, b1, HEADS, HID))
    h = jax.nn.relu(_feast(h, src, dst, W2, u2, c2, b2, HEADS, LAT))
    h = _matmul(h, Wl) + bl

    deg = jax.ops.segment_sum(ew, dst, num_segments=N)
    dinv = jnp.where(deg > 0, deg ** -0.5, 0.0)
    norm = dinv[src] * ew * dinv[dst]

    he = jax.nn.relu(_gcn(h, src, dst, norm, We1, be1))
    mu = _gcn(he, src, dst, norm, Wmu, bmu)
    logvar = _gcn(he, src, dst, norm, Wlv, blv)
    eps = jax.random.normal(jax.random.key(42), mu.shape, dtype=jnp.float32)
    z = mu + jnp.exp(0.5 * logvar) * eps
    hd = jax.nn.relu(_gcn(z, src, dst, norm, Wd1, bd1))
    recon = _gcn(hd, src, dst, norm, Wd2, bd2)
    return recon, mu, logvar


# speedup vs baseline: 4.6290x; 4.6290x over previous
import functools

import jax
import jax.numpy as jnp
from jax import lax
from jax.experimental import pallas as pl
from jax.experimental.pallas import tpu as pltpu
from jax.experimental.pallas import tpu_sc as plsc

N = 10000
E = 160000
F = 128
HID = 256
LAT = 128
OUT = 128
HEADS = 2

NS = 16
NROW_A = 640
NROW_L = N - NROW_A * (NS - 1)
E2P = 2048 * 84
NPAD = E2P - (E + N)
CHUNK = E2P // NS

ROW_BLK = 1000
NXU = 10112

_f32 = jnp.float32
_i32 = jnp.int32


def _mesh():
    return plsc.VectorSubcoreMesh(core_axis_name="c", subcore_axis_name="s")


def _rowsplit(s, fn):
    r0 = s * NROW_A

    @pl.when(s < NS - 1)
    def _():
        fn(r0, NROW_A)

    @pl.when(s == NS - 1)
    def _():
        fn(r0, NROW_L)


def _make_spmm(colsplit, bw):
    @functools.partial(
        pl.kernel,
        out_type=jax.ShapeDtypeStruct((2, N, 128), _f32),
        mesh=_mesh(),
        compiler_params=pltpu.CompilerParams(needs_layout_passes=False),
        scratch_types=[
            pltpu.VMEM_SHARED((N, 128), _f32),
            pltpu.VMEM((bw,), _i32),
            pltpu.VMEM((bw,), _i32),
            pltpu.VMEM((bw,), _f32),
            pltpu.VMEM((bw, 128), _f32),
            pltpu.SemaphoreType.DMA,
        ],
    )
    def spmm(y_hbm, src_hbm, dst_hbm, ew_hbm, zero_hbm, out_hbm,
             acc, idx_v, dst_v, ew_v, rows_v, sem):
        c = lax.axis_index("c")
        s = lax.axis_index("s")
        _rowsplit(s, lambda r0, nr: pltpu.sync_copy(
            zero_hbm.at[pl.ds(r0, nr)], acc.at[pl.ds(r0, nr)]))
        plsc.subcore_barrier()
        if colsplit:
            cn_vec = jnp.zeros((16,), _i32) + c * N
            base = s * CHUNK
            nwin = CHUNK // bw
        else:
            cn_vec = jnp.zeros((16,), _i32)
            base = s * CHUNK + c * (CHUNK // 2)
            nwin = CHUNK // bw // 2

        def window(w, carry):
            off = base + w * bw
            pltpu.sync_copy(src_hbm.at[pl.ds(off, bw)], idx_v)
            pltpu.sync_copy(dst_hbm.at[pl.ds(off, bw)], dst_v)
            pltpu.sync_copy(ew_hbm.at[pl.ds(off, bw)], ew_v)
            if colsplit:
                for j in range(bw // 16):
                    idx_v[pl.ds(j * 16, 16)] = (
                        idx_v[pl.ds(j * 16, 16)] + cn_vec)
            pltpu.async_copy(y_hbm.at[idx_v], rows_v, sem).wait()

            def scale(e, _):
                bc = plsc.load_gather(ew_v, [jnp.full((16,), e, _i32)])
                for jj in range(8):
                    rows_v[e, pl.ds(jj * 16, 16)] = (
                        rows_v[e, pl.ds(jj * 16, 16)] * bc)
                return 0

            lax.fori_loop(0, bw, scale, 0)
            pltpu.sync_copy(rows_v, acc.at[dst_v], add=True)
            return carry

        lax.fori_loop(0, nwin, window, 0)
        plsc.subcore_barrier()
        _rowsplit(s, lambda r0, nr: pltpu.sync_copy(
            acc.at[pl.ds(r0, nr)], out_hbm.at[c, pl.ds(r0, nr)]))

    return spmm


def _make_feast(colsplit, bw):
    scratch = [
        pltpu.VMEM_SHARED((N, 128), _f32),
        pltpu.VMEM((NXU,), _f32),
        pltpu.VMEM((NXU,), _f32),
        pltpu.VMEM((128,), _f32),
        pltpu.VMEM((bw,), _i32),
        pltpu.VMEM((bw,), _i32),
        pltpu.VMEM((bw,), _i32),
        pltpu.VMEM((bw,), _i32),
        pltpu.VMEM((bw,), _f32),
        pltpu.VMEM((bw,), _f32),
        pltpu.VMEM((bw,), _f32),
        pltpu.VMEM((bw, 128), _f32),
        pltpu.VMEM((bw, 128), _f32),
        pltpu.SemaphoreType.DMA,
    ]

    @functools.partial(
        pl.kernel, out_type=jax.ShapeDtypeStruct((2, N, 128), _f32),
        mesh=_mesh(),
        compiler_params=pltpu.CompilerParams(needs_layout_passes=False),
        scratch_types=scratch)
    def feast(xw_hbm, src_hbm, dst_hbm, msk_hbm, xu0_hbm, xu1_hbm,
              cpad_hbm, zero_hbm, out_hbm,
              acc, xu0_v, xu1_v, cv, src_v, dst_v, i0_v, i1_v, q0_v, q1_v,
              msk_v, r0_v, r1_v, sem):
        c = lax.axis_index("c")
        s = lax.axis_index("s")
        _rowsplit(s, lambda r0, nr: pltpu.sync_copy(
            zero_hbm.at[pl.ds(r0, nr)], acc.at[pl.ds(r0, nr)]))
        pltpu.sync_copy(xu0_hbm, xu0_v)
        pltpu.sync_copy(xu1_hbm, xu1_v)
        pltpu.sync_copy(cpad_hbm, cv)
        plsc.subcore_barrier()
        zl = jnp.zeros((16,), _i32)
        c0b = plsc.load_gather(cv, [zl])
        c1b = plsc.load_gather(cv, [zl + 1])
        if colsplit:
            cn0 = jnp.zeros((16,), _i32) + c * N
            cn1 = cn0 + 2 * N
            base = s * CHUNK
            nwin = CHUNK // bw
        else:
            cn0 = jnp.zeros((16,), _i32)
            cn1 = cn0 + N
            base = s * CHUNK + c * (CHUNK // 2)
            nwin = CHUNK // bw // 2

        def window(w, carry):
            off = base + w * bw
            pltpu.sync_copy(src_hbm.at[pl.ds(off, bw)], src_v)
            pltpu.sync_copy(dst_hbm.at[pl.ds(off, bw)], dst_v)
            pltpu.sync_copy(msk_hbm.at[pl.ds(off, bw)], msk_v)
            for j in range(bw // 16):
                sl = pl.ds(j * 16, 16)
                sv = src_v[sl]
                dv = dst_v[sl]
                x0 = (plsc.load_gather(xu0_v, [sv])
                      - plsc.load_gather(xu0_v, [dv]) + c0b)
                x1 = (plsc.load_gather(xu1_v, [sv])
                      - plsc.load_gather(xu1_v, [dv]) + c1b)
                m = jnp.maximum(x0, x1)
                e0 = jnp.exp(x0 - m)
                e1 = jnp.exp(x1 - m)
                rs = msk_v[sl] / (e0 + e1)
                q0_v[sl] = e0 * rs
                q1_v[sl] = e1 * rs
                i0_v[sl] = sv + cn0
                i1_v[sl] = sv + cn1
            pltpu.async_copy(xw_hbm.at[i0_v], r0_v, sem).wait()
            pltpu.async_copy(xw_hbm.at[i1_v], r1_v, sem).wait()

            def scale(e, _):
                ee = jnp.full((16,), e, _i32)
                b0 = plsc.load_gather(q0_v, [ee])
                b1 = plsc.load_gather(q1_v, [ee])
                for jj in range(8):
                    sl2 = pl.ds(jj * 16, 16)
                    r0_v[e, sl2] = r0_v[e, sl2] * b0 + r1_v[e, sl2] * b1
                return 0

            lax.fori_loop(0, bw, scale, 0)
            pltpu.sync_copy(r0_v, acc.at[dst_v], add=True)
            return carry

        lax.fori_loop(0, nwin, window, 0)
        plsc.subcore_barrier()
        _rowsplit(s, lambda r0, nr: pltpu.sync_copy(
            acc.at[pl.ds(r0, nr)], out_hbm.at[c, pl.ds(r0, nr)]))

    return feast


@functools.partial(
    pl.kernel, out_type=jax.ShapeDtypeStruct((2, N, 128), _f32),
    mesh=_mesh(),
    compiler_params=pltpu.CompilerParams(needs_layout_passes=False),
    scratch_types=[
        pltpu.VMEM_SHARED((N, 128), _f32),
        pltpu.VMEM((128,), _i32),
        pltpu.VMEM((128,), _f32),
        pltpu.VMEM((128,), _f32),
        pltpu.VMEM((128, 128), _f32),
    ])
def _degcnt(dst_hbm, ew_hbm, msk_hbm, zero_hbm, out_hbm,
            dcacc, dst_v, ew_v, msk_v, d_v):
    c = lax.axis_index("c")
    s = lax.axis_index("s")
    _rowsplit(s, lambda r0, nr: pltpu.sync_copy(
        zero_hbm.at[pl.ds(r0, nr)], dcacc.at[pl.ds(r0, nr)]))
    plsc.subcore_barrier()
    base = s * CHUNK + c * (CHUNK // 2)
    zv = jnp.zeros((16,), _f32)

    def window(w, carry):
        off = base + w * 128
        pltpu.sync_copy(dst_hbm.at[pl.ds(off, 128)], dst_v)
        pltpu.sync_copy(ew_hbm.at[pl.ds(off, 128)], ew_v)
        pltpu.sync_copy(msk_hbm.at[pl.ds(off, 128)], msk_v)

        def build(e, _):
            ee = jnp.full((16,), e, _i32)
            iot = lax.broadcasted_iota(_i32, (16,), 0)
            bew = plsc.load_gather(ew_v, [ee])
            bm = plsc.load_gather(msk_v, [ee])
            d_v[e, pl.ds(0, 16)] = (jnp.where(iot == 0, bew, 0.0)
                                    + jnp.where(iot == 1, bm, 0.0))
            for jj in range(1, 8):
                d_v[e, pl.ds(jj * 16, 16)] = zv
            return 0

        lax.fori_loop(0, 128, build, 0)
        pltpu.sync_copy(d_v, dcacc.at[dst_v], add=True)
        return carry

    lax.fori_loop(0, CHUNK // 128 // 2, window, 0)
    plsc.subcore_barrier()
    _rowsplit(s, lambda r0, nr: pltpu.sync_copy(
        dcacc.at[pl.ds(r0, nr)], out_hbm.at[c, pl.ds(r0, nr)]))


_spmm_cs = _make_spmm(True, 128)
_spmm_es = _make_spmm(False, 128)
_feast1 = _make_feast(True, 64)
_feast2 = _make_feast(False, 64)


def _dot(a, b):
    return jnp.dot(a, b, preferred_element_type=_f32)


def _tc_call(body, out_shapes, ins, in_specs, out_specs):
    return pl.pallas_call(
        body,
        grid=(N // ROW_BLK,),
        in_specs=in_specs,
        out_specs=out_specs,
        out_shape=out_shapes,
    )(*ins)


def _rows(k):
    return pl.BlockSpec((ROW_BLK, k), lambda i: (i, 0))


def _full(*shape):
    nd = len(shape)
    return pl.BlockSpec(shape, lambda i: (0,) * nd)


def _stk(nsplit, k):
    return pl.BlockSpec((nsplit, ROW_BLK, k), lambda i: (0, i, 0))


def _tc_feast_pre(x, W, upad):
    def body(x_ref, w_ref, u_ref, o1_ref, o2_ref):
        xb = x_ref[...]
        for j in range(4):
            o1_ref[j] = _dot(xb, w_ref[:, j * 128:(j + 1) * 128])
        o2_ref[...] = _dot(xb, u_ref[...])

    k = x.shape[1]
    return _tc_call(
        body,
        (jax.ShapeDtypeStruct((4, N, 128), _f32),
         jax.ShapeDtypeStruct((N, 128), _f32)),
        (x, W, upad),
        [_rows(k), _full(k, 512), _full(k, 128)],
        (_stk(4, 128), _rows(128)),
    )


def _tc_feast_mid(acc1, cinv2d, b1p, W2, u2pad):
    def body(a_ref, ci_ref, b_ref, w_ref, u_ref, o1_ref, o2_ref):
        ci = ci_ref[...]
        t0 = jnp.maximum(a_ref[0] * ci + b_ref[0:1, :128], 0.0)
        t1 = jnp.maximum(a_ref[1] * ci + b_ref[0:1, 128:], 0.0)
        h1 = jnp.concatenate([t0, t1], axis=1)
        for j in range(2):
            o1_ref[j] = _dot(h1, w_ref[:, j * 128:(j + 1) * 128])
        o2_ref[...] = _dot(h1, u_ref[...])

    return _tc_call(
        body,
        (jax.ShapeDtypeStruct((2, N, 128), _f32),
         jax.ShapeDtypeStruct((N, 128), _f32)),
        (acc1, cinv2d, b1p, W2, u2pad),
        [_stk(2, 128), _rows(128), _full(8, 256), _full(256, 256),
         _full(256, 128)],
        (_stk(2, 128), _rows(128)),
    )


def _tc_gcn_pre(acc2, cinv2d, b2p, Wl, blp, We1, dinv2d):
    def body(a_ref, ci_ref, b2_ref, wl_ref, bl_ref, we_ref, di_ref, o_ref):
        t = a_ref[0] + a_ref[1]
        t = jnp.maximum(t * ci_ref[...] + b2_ref[0:1], 0.0)
        h = _dot(t, wl_ref[...]) + bl_ref[0:1]
        di = di_ref[...]
        for j in range(2):
            o_ref[j] = di * _dot(h, we_ref[:, j * 128:(j + 1) * 128])

    return _tc_call(
        body,
        jax.ShapeDtypeStruct((2, N, 128), _f32),
        (acc2, cinv2d, b2p, Wl, blp, We1, dinv2d),
        [_stk(2, 128), _rows(128), _full(8, 128), _full(128, 128),
         _full(8, 128), _full(128, 256), _rows(128)],
        _stk(2, 128),
    )


def _tc_gcn_mid(agg, dinv2d, bp, Wn):
    def body(a_ref, di_ref, b_ref, w_ref, o_ref):
        di = di_ref[...]
        t0 = jnp.maximum(a_ref[0] * di + b_ref[0:1, :128], 0.0)
        t1 = jnp.maximum(a_ref[1] * di + b_ref[0:1, 128:], 0.0)
        t = jnp.concatenate([t0, t1], axis=1)
        for j in range(2):
            o_ref[j] = di * _dot(t, w_ref[:, j * 128:(j + 1) * 128])

    return _tc_call(
        body,
        jax.ShapeDtypeStruct((2, N, 128), _f32),
        (agg, dinv2d, bp, Wn),
        [_stk(2, 128), _rows(128), _full(8, 256), _full(256, 256)],
        _stk(2, 128),
    )


def _tc_gcn_last(agg, dinv2d, bp, Wn):
    def body(a_ref, di_ref, b_ref, w_ref, o_ref):
        di = di_ref[...]
        t0 = jnp.maximum(a_ref[0] * di + b_ref[0:1, :128], 0.0)
        t1 = jnp.maximum(a_ref[1] * di + b_ref[0:1, 128:], 0.0)
        t = jnp.concatenate([t0, t1], axis=1)
        o_ref[...] = di * _dot(t, w_ref[...])

    return _tc_call(
        body,
        jax.ShapeDtypeStruct((N, 128), _f32),
        (agg, dinv2d, bp, Wn),
        [_stk(2, 128), _rows(128), _full(8, 256), _full(256, 128)],
        _rows(128),
    )


def _tc_z(agg2, dinv2d, bmup, blvp, eps, Wd1):
    def body(a_ref, di_ref, bm_ref, bl_ref, e_ref, w_ref,
             mu_ref, lv_ref, o_ref):
        di = di_ref[...]
        mu = a_ref[0] * di + bm_ref[0:1]
        lv = a_ref[1] * di + bl_ref[0:1]
        mu_ref[...] = mu
        lv_ref[...] = lv
        z = mu + jnp.exp(0.5 * lv) * e_ref[...]
        for j in range(2):
            o_ref[j] = di * _dot(z, w_ref[:, j * 128:(j + 1) * 128])

    return _tc_call(
        body,
        (jax.ShapeDtypeStruct((N, 128), _f32),
         jax.ShapeDtypeStruct((N, 128), _f32),
         jax.ShapeDtypeStruct((2, N, 128), _f32)),
        (agg2, dinv2d, bmup, blvp, eps, Wd1),
        [_stk(2, 128), _rows(128), _full(8, 128), _full(8, 128),
         _rows(128), _full(128, 256)],
        (_rows(128), _rows(128), _stk(2, 128)),
    )


def _tc_final(agg4, dinv2d, bd2p):
    def body(a_ref, di_ref, b_ref, o_ref):
        t = a_ref[0] + a_ref[1]
        o_ref[...] = t * di_ref[...] + b_ref[0:1]

    return _tc_call(
        body,
        jax.ShapeDtypeStruct((N, 128), _f32),
        (agg4, dinv2d, bd2p),
        [_stk(2, 128), _rows(128), _full(8, 128)],
        _rows(128),
    )


def kernel(x, edge_index, edge_weight, W1, u1, c1, b1, W2, u2, c2, b2, Wl, bl,
           We1, be1, Wmu, bmu, Wlv, blv, Wd1, bd1, Wd2, bd2):
    idt = edge_index.dtype
    loop = jnp.arange(N, dtype=idt)
    padi = jnp.arange(NPAD, dtype=idt) % N
    src = jnp.concatenate([edge_index[0], loop, padi])
    dst = jnp.concatenate([edge_index[1], loop, padi])
    ew = jnp.concatenate([edge_weight, jnp.ones((N,), _f32),
                          jnp.zeros((NPAD,), _f32)])
    msk = jnp.concatenate([jnp.ones((E + N,), _f32),
                           jnp.zeros((NPAD,), _f32)])
    zeros128 = jnp.zeros((N, 128), _f32)

    def _padxu(v):
        return jnp.pad(v, (0, NXU - N))

    def pad_u(u):
        return jnp.zeros((u.shape[0], 128), _f32).at[:, :HEADS].set(u)

    def pad_b(b):
        return jnp.broadcast_to(b[None, :], (8, b.shape[0]))

    cpad = lambda cc: jnp.zeros((128,), _f32).at[:HEADS].set(cc)

    dc = _degcnt(dst, ew, msk, zeros128)
    deg = dc[0, :, 0] + dc[1, :, 0]
    cnt = dc[0, :, 1] + dc[1, :, 1]
    cinv2d = jnp.broadcast_to((1.0 / jnp.clip(cnt, 1.0, None))[:, None],
                              (N, 128))
    dinv = jnp.where(deg > 0, lax.rsqrt(deg), 0.0)
    dinv2d = jnp.broadcast_to(dinv[:, None], (N, 128))

    xw1stk, xu1p = _tc_feast_pre(x, W1, pad_u(u1))
    acc1 = _feast1(
        xw1stk.reshape(4 * N, 128), src, dst, msk,
        _padxu(xu1p[:, 0]), _padxu(xu1p[:, 1]), cpad(c1), zeros128)

    xw2stk, xu2p = _tc_feast_mid(acc1, cinv2d, pad_b(b1), W2, pad_u(u2))
    acc2 = _feast2(
        xw2stk.reshape(2 * N, 128), src, dst, msk,
        _padxu(xu2p[:, 0]), _padxu(xu2p[:, 1]), cpad(c2), zeros128)

    y1 = _tc_gcn_pre(acc2, cinv2d, pad_b(b2), Wl, pad_b(bl), We1, dinv2d)
    agg1 = _spmm_cs(y1.reshape(2 * N, 128), src, dst, ew, zeros128)
    Wml = jnp.concatenate([Wmu, Wlv], axis=1)
    y2 = _tc_gcn_mid(agg1, dinv2d, pad_b(be1), Wml)
    agg2 = _spmm_cs(y2.reshape(2 * N, 128), src, dst, ew, zeros128)
    eps = jax.random.normal(jax.random.key(42), (N, LAT), dtype=_f32)
    mu, logvar, y3 = _tc_z(agg2, dinv2d, pad_b(bmu), pad_b(blv), eps, Wd1)
    agg3 = _spmm_cs(y3.reshape(2 * N, 128), src, dst, ew, zeros128)
    y4 = _tc_gcn_last(agg3, dinv2d, pad_b(bd1), Wd2)
    agg4 = _spmm_es(y4, src, dst, ew, zeros128)
    recon = _tc_final(agg4, dinv2d, bd2p=pad_b(bd2))
    return recon, mu, logvar


# speedup vs baseline: 5.7746x; 1.2475x over previous
import functools

import jax
import jax.numpy as jnp
from jax import lax
from jax.experimental import pallas as pl
from jax.experimental.pallas import tpu as pltpu
from jax.experimental.pallas import tpu_sc as plsc

N = 10000
E = 160000
F = 128
HID = 256
LAT = 128
OUT = 128
HEADS = 2

NS = 16
NROW_A = 640
NROW_L = N - NROW_A * (NS - 1)
E2P = 2048 * 84
NPAD = E2P - (E + N)
CHUNK = E2P // NS

ROW_BLK = 1000
NXU = 10112

_f32 = jnp.float32
_i32 = jnp.int32


def _mesh():
    return plsc.VectorSubcoreMesh(core_axis_name="c", subcore_axis_name="s")


def _rowsplit(s, fn):
    r0 = s * NROW_A

    @pl.when(s < NS - 1)
    def _():
        fn(r0, NROW_A)

    @pl.when(s == NS - 1)
    def _():
        fn(r0, NROW_L)


def _make_spmm(colsplit, bw):
    @functools.partial(
        pl.kernel,
        out_type=jax.ShapeDtypeStruct((2, N, 128), _f32),
        mesh=_mesh(),
        compiler_params=pltpu.CompilerParams(needs_layout_passes=False),
        scratch_types=[
            pltpu.VMEM_SHARED((N, 128), _f32),
            pltpu.VMEM((bw,), _i32),
            pltpu.VMEM((bw,), _i32),
            pltpu.VMEM((bw,), _f32),
            pltpu.VMEM((bw, 128), _f32),
            pltpu.SemaphoreType.DMA,
        ],
    )
    def spmm(y_hbm, src_hbm, dst_hbm, ew_hbm, zero_hbm, out_hbm,
             acc, idx_v, dst_v, ew_v, rows_v, sem):
        c = lax.axis_index("c")
        s = lax.axis_index("s")
        _rowsplit(s, lambda r0, nr: pltpu.sync_copy(
            zero_hbm.at[pl.ds(r0, nr)], acc.at[pl.ds(r0, nr)]))
        plsc.subcore_barrier()
        if colsplit:
            cn_vec = jnp.zeros((16,), _i32) + c * N
            base = s * CHUNK
            nwin = CHUNK // bw
        else:
            cn_vec = jnp.zeros((16,), _i32)
            base = s * CHUNK + c * (CHUNK // 2)
            nwin = CHUNK // bw // 2

        def window(w, carry):
            off = base + w * bw
            pltpu.sync_copy(src_hbm.at[pl.ds(off, bw)], idx_v)
            pltpu.sync_copy(dst_hbm.at[pl.ds(off, bw)], dst_v)
            pltpu.sync_copy(ew_hbm.at[pl.ds(off, bw)], ew_v)
            if colsplit:
                for j in range(bw // 16):
                    idx_v[pl.ds(j * 16, 16)] = (
                        idx_v[pl.ds(j * 16, 16)] + cn_vec)
            pltpu.async_copy(y_hbm.at[idx_v], rows_v, sem).wait()

            def scale(g, _):
                g16 = pl.multiple_of(g * 16, 16)
                wv = ew_v[pl.ds(g16, 16)]
                for e in range(16):
                    bc = jnp.broadcast_to(wv[e], (16,))
                    for jj in range(8):
                        rows_v[g16 + e, pl.ds(jj * 16, 16)] = (
                            rows_v[g16 + e, pl.ds(jj * 16, 16)] * bc)
                return 0

            lax.fori_loop(0, bw // 16, scale, 0)
            pltpu.sync_copy(rows_v, acc.at[dst_v], add=True)
            return carry

        lax.fori_loop(0, nwin, window, 0)
        plsc.subcore_barrier()
        _rowsplit(s, lambda r0, nr: pltpu.sync_copy(
            acc.at[pl.ds(r0, nr)], out_hbm.at[c, pl.ds(r0, nr)]))

    return spmm


def _make_feast(colsplit, bw):
    scratch = [
        pltpu.VMEM_SHARED((N, 128), _f32),
        pltpu.VMEM((NXU,), _f32),
        pltpu.VMEM((NXU,), _f32),
        pltpu.VMEM((128,), _f32),
        pltpu.VMEM((bw,), _i32),
        pltpu.VMEM((bw,), _i32),
        pltpu.VMEM((bw,), _i32),
        pltpu.VMEM((bw,), _i32),
        pltpu.VMEM((bw,), _f32),
        pltpu.VMEM((bw,), _f32),
        pltpu.VMEM((bw,), _f32),
        pltpu.VMEM((bw, 128), _f32),
        pltpu.VMEM((bw, 128), _f32),
        pltpu.SemaphoreType.DMA,
    ]

    @functools.partial(
        pl.kernel, out_type=jax.ShapeDtypeStruct((2, N, 128), _f32),
        mesh=_mesh(),
        compiler_params=pltpu.CompilerParams(needs_layout_passes=False),
        scratch_types=scratch)
    def feast(xw_hbm, src_hbm, dst_hbm, msk_hbm, xu0_hbm, xu1_hbm,
              cpad_hbm, zero_hbm, out_hbm,
              acc, xu0_v, xu1_v, cv, src_v, dst_v, i0_v, i1_v, q0_v, q1_v,
              msk_v, r0_v, r1_v, sem):
        c = lax.axis_index("c")
        s = lax.axis_index("s")
        _rowsplit(s, lambda r0, nr: pltpu.sync_copy(
            zero_hbm.at[pl.ds(r0, nr)], acc.at[pl.ds(r0, nr)]))
        pltpu.sync_copy(xu0_hbm, xu0_v)
        pltpu.sync_copy(xu1_hbm, xu1_v)
        pltpu.sync_copy(cpad_hbm, cv)
        plsc.subcore_barrier()
        zl = jnp.zeros((16,), _i32)
        c0b = plsc.load_gather(cv, [zl])
        c1b = plsc.load_gather(cv, [zl + 1])
        if colsplit:
            cn0 = jnp.zeros((16,), _i32) + c * N
            cn1 = cn0 + 2 * N
            base = s * CHUNK
            nwin = CHUNK // bw
        else:
            cn0 = jnp.zeros((16,), _i32)
            cn1 = cn0 + N
            base = s * CHUNK + c * (CHUNK // 2)
            nwin = CHUNK // bw // 2

        def window(w, carry):
            off = base + w * bw
            pltpu.sync_copy(src_hbm.at[pl.ds(off, bw)], src_v)
            pltpu.sync_copy(dst_hbm.at[pl.ds(off, bw)], dst_v)
            pltpu.sync_copy(msk_hbm.at[pl.ds(off, bw)], msk_v)
            for j in range(bw // 16):
                sl = pl.ds(j * 16, 16)
                sv = src_v[sl]
                dv = dst_v[sl]
                x0 = (plsc.load_gather(xu0_v, [sv])
                      - plsc.load_gather(xu0_v, [dv]) + c0b)
                x1 = (plsc.load_gather(xu1_v, [sv])
                      - plsc.load_gather(xu1_v, [dv]) + c1b)
                m = jnp.maximum(x0, x1)
                e0 = jnp.exp(x0 - m)
                e1 = jnp.exp(x1 - m)
                rs = msk_v[sl] / (e0 + e1)
                q0_v[sl] = e0 * rs
                q1_v[sl] = e1 * rs
                i0_v[sl] = sv + cn0
                i1_v[sl] = sv + cn1
            pltpu.async_copy(xw_hbm.at[i0_v], r0_v, sem).wait()
            pltpu.async_copy(xw_hbm.at[i1_v], r1_v, sem).wait()

            def scale(g, _):
                g16 = pl.multiple_of(g * 16, 16)
                v0 = q0_v[pl.ds(g16, 16)]
                v1 = q1_v[pl.ds(g16, 16)]
                for e in range(16):
                    b0 = jnp.broadcast_to(v0[e], (16,))
                    b1 = jnp.broadcast_to(v1[e], (16,))
                    for jj in range(8):
                        sl2 = pl.ds(jj * 16, 16)
                        r0_v[g16 + e, sl2] = (r0_v[g16 + e, sl2] * b0
                                              + r1_v[g16 + e, sl2] * b1)
                return 0

            lax.fori_loop(0, bw // 16, scale, 0)
            pltpu.sync_copy(r0_v, acc.at[dst_v], add=True)
            return carry

        lax.fori_loop(0, nwin, window, 0)
        plsc.subcore_barrier()
        _rowsplit(s, lambda r0, nr: pltpu.sync_copy(
            acc.at[pl.ds(r0, nr)], out_hbm.at[c, pl.ds(r0, nr)]))

    return feast


@functools.partial(
    pl.kernel, out_type=jax.ShapeDtypeStruct((2, N, 128), _f32),
    mesh=_mesh(),
    compiler_params=pltpu.CompilerParams(needs_layout_passes=False),
    scratch_types=[
        pltpu.VMEM_SHARED((N, 128), _f32),
        pltpu.VMEM((128,), _i32),
        pltpu.VMEM((128,), _f32),
        pltpu.VMEM((128,), _f32),
        pltpu.VMEM((128, 128), _f32),
    ])
def _degcnt(dst_hbm, ew_hbm, msk_hbm, zero_hbm, out_hbm,
            dcacc, dst_v, ew_v, msk_v, d_v):
    c = lax.axis_index("c")
    s = lax.axis_index("s")
    _rowsplit(s, lambda r0, nr: pltpu.sync_copy(
        zero_hbm.at[pl.ds(r0, nr)], dcacc.at[pl.ds(r0, nr)]))
    plsc.subcore_barrier()
    base = s * CHUNK + c * (CHUNK // 2)
    zv = jnp.zeros((16,), _f32)

    def window(w, carry):
        off = base + w * 128
        pltpu.sync_copy(dst_hbm.at[pl.ds(off, 128)], dst_v)
        pltpu.sync_copy(ew_hbm.at[pl.ds(off, 128)], ew_v)
        pltpu.sync_copy(msk_hbm.at[pl.ds(off, 128)], msk_v)

        iot = lax.broadcasted_iota(_i32, (16,), 0)

        def build(g, _):
            g16 = pl.multiple_of(g * 16, 16)
            wv = ew_v[pl.ds(g16, 16)]
            mv = msk_v[pl.ds(g16, 16)]
            for e in range(16):
                bew = jnp.broadcast_to(wv[e], (16,))
                bm = jnp.broadcast_to(mv[e], (16,))
                d_v[g16 + e, pl.ds(0, 16)] = (
                    jnp.where(iot == 0, bew, 0.0)
                    + jnp.where(iot == 1, bm, 0.0))
                for jj in range(1, 8):
                    d_v[g16 + e, pl.ds(jj * 16, 16)] = zv
            return 0

        lax.fori_loop(0, 8, build, 0)
        pltpu.sync_copy(d_v, dcacc.at[dst_v], add=True)
        return carry

    lax.fori_loop(0, CHUNK // 128 // 2, window, 0)
    plsc.subcore_barrier()
    _rowsplit(s, lambda r0, nr: pltpu.sync_copy(
        dcacc.at[pl.ds(r0, nr)], out_hbm.at[c, pl.ds(r0, nr)]))


_spmm_cs = _make_spmm(True, 128)
_spmm_es = _make_spmm(False, 128)
_feast1 = _make_feast(True, 64)
_feast2 = _make_feast(False, 64)


def _dot(a, b):
    return jnp.dot(a, b, preferred_element_type=_f32)


def _tc_call(body, out_shapes, ins, in_specs, out_specs):
    return pl.pallas_call(
        body,
        grid=(N // ROW_BLK,),
        in_specs=in_specs,
        out_specs=out_specs,
        out_shape=out_shapes,
    )(*ins)


def _rows(k):
    return pl.BlockSpec((ROW_BLK, k), lambda i: (i, 0))


def _full(*shape):
    nd = len(shape)
    return pl.BlockSpec(shape, lambda i: (0,) * nd)


def _stk(nsplit, k):
    return pl.BlockSpec((nsplit, ROW_BLK, k), lambda i: (0, i, 0))


def _tc_feast_pre(x, W, upad):
    def body(x_ref, w_ref, u_ref, o1_ref, o2_ref):
        xb = x_ref[...]
        for j in range(4):
            o1_ref[j] = _dot(xb, w_ref[:, j * 128:(j + 1) * 128])
        o2_ref[...] = _dot(xb, u_ref[...])

    k = x.shape[1]
    return _tc_call(
        body,
        (jax.ShapeDtypeStruct((4, N, 128), _f32),
         jax.ShapeDtypeStruct((N, 128), _f32)),
        (x, W, upad),
        [_rows(k), _full(k, 512), _full(k, 128)],
        (_stk(4, 128), _rows(128)),
    )


def _tc_feast_mid(acc1, cinv2d, b1p, W2, u2pad):
    def body(a_ref, ci_ref, b_ref, w_ref, u_ref, o1_ref, o2_ref):
        ci = ci_ref[...]
        t0 = jnp.maximum(a_ref[0] * ci + b_ref[0:1, :128], 0.0)
        t1 = jnp.maximum(a_ref[1] * ci + b_ref[0:1, 128:], 0.0)
        h1 = jnp.concatenate([t0, t1], axis=1)
        for j in range(2):
            o1_ref[j] = _dot(h1, w_ref[:, j * 128:(j + 1) * 128])
        o2_ref[...] = _dot(h1, u_ref[...])

    return _tc_call(
        body,
        (jax.ShapeDtypeStruct((2, N, 128), _f32),
         jax.ShapeDtypeStruct((N, 128), _f32)),
        (acc1, cinv2d, b1p, W2, u2pad),
        [_stk(2, 128), _rows(128), _full(8, 256), _full(256, 256),
         _full(256, 128)],
        (_stk(2, 128), _rows(128)),
    )


def _tc_gcn_pre(acc2, cinv2d, b2p, Wl, blp, We1, dinv2d):
    def body(a_ref, ci_ref, b2_ref, wl_ref, bl_ref, we_ref, di_ref, o_ref):
        t = a_ref[0] + a_ref[1]
        t = jnp.maximum(t * ci_ref[...] + b2_ref[0:1], 0.0)
        h = _dot(t, wl_ref[...]) + bl_ref[0:1]
        di = di_ref[...]
        for j in range(2):
            o_ref[j] = di * _dot(h, we_ref[:, j * 128:(j + 1) * 128])

    return _tc_call(
        body,
        jax.ShapeDtypeStruct((2, N, 128), _f32),
        (acc2, cinv2d, b2p, Wl, blp, We1, dinv2d),
        [_stk(2, 128), _rows(128), _full(8, 128), _full(128, 128),
         _full(8, 128), _full(128, 256), _rows(128)],
        _stk(2, 128),
    )


def _tc_gcn_mid(agg, dinv2d, bp, Wn):
    def body(a_ref, di_ref, b_ref, w_ref, o_ref):
        di = di_ref[...]
        t0 = jnp.maximum(a_ref[0] * di + b_ref[0:1, :128], 0.0)
        t1 = jnp.maximum(a_ref[1] * di + b_ref[0:1, 128:], 0.0)
        t = jnp.concatenate([t0, t1], axis=1)
        for j in range(2):
            o_ref[j] = di * _dot(t, w_ref[:, j * 128:(j + 1) * 128])

    return _tc_call(
        body,
        jax.ShapeDtypeStruct((2, N, 128), _f32),
        (agg, dinv2d, bp, Wn),
        [_stk(2, 128), _rows(128), _full(8, 256), _full(256, 256)],
        _stk(2, 128),
    )


def _tc_gcn_last(agg, dinv2d, bp, Wn):
    def body(a_ref, di_ref, b_ref, w_ref, o_ref):
        di = di_ref[...]
        t0 = jnp.maximum(a_ref[0] * di + b_ref[0:1, :128], 0.0)
        t1 = jnp.maximum(a_ref[1] * di + b_ref[0:1, 128:], 0.0)
        t = jnp.concatenate([t0, t1], axis=1)
        o_ref[...] = di * _dot(t, w_ref[...])

    return _tc_call(
        body,
        jax.ShapeDtypeStruct((N, 128), _f32),
        (agg, dinv2d, bp, Wn),
        [_stk(2, 128), _rows(128), _full(8, 256), _full(256, 128)],
        _rows(128),
    )


def _tc_z(agg2, dinv2d, bmup, blvp, eps, Wd1):
    def body(a_ref, di_ref, bm_ref, bl_ref, e_ref, w_ref,
             mu_ref, lv_ref, o_ref):
        di = di_ref[...]
        mu = a_ref[0] * di + bm_ref[0:1]
        lv = a_ref[1] * di + bl_ref[0:1]
        mu_ref[...] = mu
        lv_ref[...] = lv
        z = mu + jnp.exp(0.5 * lv) * e_ref[...]
        for j in range(2):
            o_ref[j] = di * _dot(z, w_ref[:, j * 128:(j + 1) * 128])

    return _tc_call(
        body,
        (jax.ShapeDtypeStruct((N, 128), _f32),
         jax.ShapeDtypeStruct((N, 128), _f32),
         jax.ShapeDtypeStruct((2, N, 128), _f32)),
        (agg2, dinv2d, bmup, blvp, eps, Wd1),
        [_stk(2, 128), _rows(128), _full(8, 128), _full(8, 128),
         _rows(128), _full(128, 256)],
        (_rows(128), _rows(128), _stk(2, 128)),
    )


def _tc_final(agg4, dinv2d, bd2p):
    def body(a_ref, di_ref, b_ref, o_ref):
        t = a_ref[0] + a_ref[1]
        o_ref[...] = t * di_ref[...] + b_ref[0:1]

    return _tc_call(
        body,
        jax.ShapeDtypeStruct((N, 128), _f32),
        (agg4, dinv2d, bd2p),
        [_stk(2, 128), _rows(128), _full(8, 128)],
        _rows(128),
    )


def kernel(x, edge_index, edge_weight, W1, u1, c1, b1, W2, u2, c2, b2, Wl, bl,
           We1, be1, Wmu, bmu, Wlv, blv, Wd1, bd1, Wd2, bd2):
    idt = edge_index.dtype
    loop = jnp.arange(N, dtype=idt)
    padi = jnp.arange(NPAD, dtype=idt) % N
    src = jnp.concatenate([edge_index[0], loop, padi])
    dst = jnp.concatenate([edge_index[1], loop, padi])
    ew = jnp.concatenate([edge_weight, jnp.ones((N,), _f32),
                          jnp.zeros((NPAD,), _f32)])
    msk = jnp.concatenate([jnp.ones((E + N,), _f32),
                           jnp.zeros((NPAD,), _f32)])
    zeros128 = jnp.zeros((N, 128), _f32)

    def _padxu(v):
        return jnp.pad(v, (0, NXU - N))

    def pad_u(u):
        return jnp.zeros((u.shape[0], 128), _f32).at[:, :HEADS].set(u)

    def pad_b(b):
        return jnp.broadcast_to(b[None, :], (8, b.shape[0]))

    cpad = lambda cc: jnp.zeros((128,), _f32).at[:HEADS].set(cc)

    dc = _degcnt(dst, ew, msk, zeros128)
    deg = dc[0, :, 0] + dc[1, :, 0]
    cnt = dc[0, :, 1] + dc[1, :, 1]
    cinv2d = jnp.broadcast_to((1.0 / jnp.clip(cnt, 1.0, None))[:, None],
                              (N, 128))
    dinv = jnp.where(deg > 0, lax.rsqrt(deg), 0.0)
    dinv2d = jnp.broadcast_to(dinv[:, None], (N, 128))

    xw1stk, xu1p = _tc_feast_pre(x, W1, pad_u(u1))
    acc1 = _feast1(
        xw1stk.reshape(4 * N, 128), src, dst, msk,
        _padxu(xu1p[:, 0]), _padxu(xu1p[:, 1]), cpad(c1), zeros128)

    xw2stk, xu2p = _tc_feast_mid(acc1, cinv2d, pad_b(b1), W2, pad_u(u2))
    acc2 = _feast2(
        xw2stk.reshape(2 * N, 128), src, dst, msk,
        _padxu(xu2p[:, 0]), _padxu(xu2p[:, 1]), cpad(c2), zeros128)

    y1 = _tc_gcn_pre(acc2, cinv2d, pad_b(b2), Wl, pad_b(bl), We1, dinv2d)
    agg1 = _spmm_cs(y1.reshape(2 * N, 128), src, dst, ew, zeros128)
    Wml = jnp.concatenate([Wmu, Wlv], axis=1)
    y2 = _tc_gcn_mid(agg1, dinv2d, pad_b(be1), Wml)
    agg2 = _spmm_cs(y2.reshape(2 * N, 128), src, dst, ew, zeros128)
    eps = jax.random.normal(jax.random.key(42), (N, LAT), dtype=_f32)
    mu, logvar, y3 = _tc_z(agg2, dinv2d, pad_b(bmu), pad_b(blv), eps, Wd1)
    agg3 = _spmm_cs(y3.reshape(2 * N, 128), src, dst, ew, zeros128)
    y4 = _tc_gcn_last(agg3, dinv2d, pad_b(bd1), Wd2)
    agg4 = _spmm_es(y4, src, dst, ew, zeros128)
    recon = _tc_final(agg4, dinv2d, bd2p=pad_b(bd2))
    return recon, mu, logvar


# speedup vs baseline: 6.8353x; 1.1837x over previous
import functools

import jax
import jax.numpy as jnp
from jax import lax
from jax.experimental import pallas as pl
from jax.experimental.pallas import tpu as pltpu
from jax.experimental.pallas import tpu_sc as plsc

N = 10000
E = 160000
F = 128
HID = 256
LAT = 128
OUT = 128
HEADS = 2

NS = 16
NROW_A = 640
NROW_L = N - NROW_A * (NS - 1)
E2P = 2048 * 96
NPAD = E2P - (E + N)
CHUNK = E2P // NS

ROW_BLK = 1000
NXU = 10112

_f32 = jnp.float32
_i32 = jnp.int32


def _mesh():
    return plsc.VectorSubcoreMesh(core_axis_name="c", subcore_axis_name="s")


def _rowsplit(s, fn):
    r0 = s * NROW_A

    @pl.when(s < NS - 1)
    def _():
        fn(r0, NROW_A)

    @pl.when(s == NS - 1)
    def _():
        fn(r0, NROW_L)


def _make_spmm(colsplit):
    BW = 128
    SEGW = 16
    nseg = (CHUNK // BW) // SEGW if colsplit else (CHUNK // BW) // SEGW // 2

    @functools.partial(
        pl.kernel,
        out_type=jax.ShapeDtypeStruct((2, N, 128), _f32),
        mesh=_mesh(),
        compiler_params=pltpu.CompilerParams(needs_layout_passes=False),
        scratch_types=[
            pltpu.VMEM_SHARED((N, 128), _f32),
            pltpu.VMEM((SEGW, BW), _i32),
            pltpu.VMEM((SEGW, BW), _i32),
            pltpu.VMEM((SEGW, BW), _f32),
            pltpu.VMEM((BW, 128), _f32),
            pltpu.VMEM((BW, 128), _f32),
            pltpu.SemaphoreType.DMA,
            pltpu.SemaphoreType.DMA,
        ],
    )
    def spmm(y_hbm, src_hbm, dst_hbm, ew_hbm, zero_hbm, out_hbm,
             acc, sidx, sdst, sew, rows0, rows1, sem0, sem1):
        c = lax.axis_index("c")
        s = lax.axis_index("s")
        _rowsplit(s, lambda r0, nr: pltpu.sync_copy(
            zero_hbm.at[pl.ds(r0, nr)], acc.at[pl.ds(r0, nr)]))
        plsc.subcore_barrier()
        if colsplit:
            cn_vec = jnp.zeros((16,), _i32) + c * N
            base_w = s * (CHUNK // BW)
        else:
            cn_vec = jnp.zeros((16,), _i32)
            base_w = s * (CHUNK // BW) + c * (CHUNK // BW // 2)

        def stage(seg):
            w0 = base_w + seg * SEGW
            pltpu.sync_copy(src_hbm.at[pl.ds(w0, SEGW)], sidx)
            pltpu.sync_copy(dst_hbm.at[pl.ds(w0, SEGW)], sdst)
            pltpu.sync_copy(ew_hbm.at[pl.ds(w0, SEGW)], sew)
            if colsplit:
                def addcn(r, _):
                    for j in range(BW // 16):
                        sidx[r, pl.ds(j * 16, 16)] = (
                            sidx[r, pl.ds(j * 16, 16)] + cn_vec)
                    return 0
                lax.fori_loop(0, SEGW, addcn, 0)

        def gather(w, buf, sem):
            pltpu.async_copy(y_hbm.at[sidx.at[w]], buf, sem)

        def gwait(w, buf, sem):
            pltpu.make_async_copy(y_hbm.at[sidx.at[w]], buf, sem).wait()

        def scale_scatter(w, buf):
            def scale(g, _):
                g16 = pl.multiple_of(g * 16, 16)
                wv = sew[w, pl.ds(g16, 16)]
                for e in range(16):
                    bc = jnp.broadcast_to(wv[e], (16,))
                    for jj in range(8):
                        buf[g16 + e, pl.ds(jj * 16, 16)] = (
                            buf[g16 + e, pl.ds(jj * 16, 16)] * bc)
                return 0

            lax.fori_loop(0, BW // 16, scale, 0)
            pltpu.sync_copy(buf, acc.at[sdst.at[w]], add=True)

        def run_segment(_seg, carry):
            stage(_seg)
            gather(0, rows0, sem0)
            gather(1, rows1, sem1)

            def pair(k, _):
                a = 2 * k
                gwait(a, rows0, sem0)
                scale_scatter(a, rows0)
                gather(a + 2, rows0, sem0)
                gwait(a + 1, rows1, sem1)
                scale_scatter(a + 1, rows1)
                gather(a + 3, rows1, sem1)
                return 0

            lax.fori_loop(0, SEGW // 2 - 1, pair, 0)
            gwait(SEGW - 2, rows0, sem0)
            scale_scatter(SEGW - 2, rows0)
            gwait(SEGW - 1, rows1, sem1)
            scale_scatter(SEGW - 1, rows1)
            return carry

        lax.fori_loop(0, nseg, run_segment, 0)
        plsc.subcore_barrier()
        _rowsplit(s, lambda r0, nr: pltpu.sync_copy(
            acc.at[pl.ds(r0, nr)], out_hbm.at[c, pl.ds(r0, nr)]))

    return spmm


def _make_feast(colsplit, bw):
    scratch = [
        pltpu.VMEM_SHARED((N, 128), _f32),
        pltpu.VMEM((NXU,), _f32),
        pltpu.VMEM((NXU,), _f32),
        pltpu.VMEM((128,), _f32),
        pltpu.VMEM((bw,), _i32),
        pltpu.VMEM((bw,), _i32),
        pltpu.VMEM((bw,), _i32),
        pltpu.VMEM((bw,), _i32),
        pltpu.VMEM((bw,), _f32),
        pltpu.VMEM((bw,), _f32),
        pltpu.VMEM((bw,), _f32),
        pltpu.VMEM((bw, 128), _f32),
        pltpu.VMEM((bw, 128), _f32),
        pltpu.SemaphoreType.DMA,
    ]

    @functools.partial(
        pl.kernel, out_type=jax.ShapeDtypeStruct((2, N, 128), _f32),
        mesh=_mesh(),
        compiler_params=pltpu.CompilerParams(needs_layout_passes=False),
        scratch_types=scratch)
    def feast(xw_hbm, src_hbm, dst_hbm, msk_hbm, xu0_hbm, xu1_hbm,
              cpad_hbm, zero_hbm, out_hbm,
              acc, xu0_v, xu1_v, cv, src_v, dst_v, i0_v, i1_v, q0_v, q1_v,
              msk_v, r0_v, r1_v, sem):
        c = lax.axis_index("c")
        s = lax.axis_index("s")
        _rowsplit(s, lambda r0, nr: pltpu.sync_copy(
            zero_hbm.at[pl.ds(r0, nr)], acc.at[pl.ds(r0, nr)]))
        pltpu.sync_copy(xu0_hbm, xu0_v)
        pltpu.sync_copy(xu1_hbm, xu1_v)
        pltpu.sync_copy(cpad_hbm, cv)
        plsc.subcore_barrier()
        zl = jnp.zeros((16,), _i32)
        c0b = plsc.load_gather(cv, [zl])
        c1b = plsc.load_gather(cv, [zl + 1])
        if colsplit:
            cn0 = jnp.zeros((16,), _i32) + c * N
            cn1 = cn0 + 2 * N
            base = s * CHUNK
            nwin = CHUNK // bw
        else:
            cn0 = jnp.zeros((16,), _i32)
            cn1 = cn0 + N
            base = s * CHUNK + c * (CHUNK // 2)
            nwin = CHUNK // bw // 2

        def window(w, carry):
            off = base + w * bw
            pltpu.sync_copy(src_hbm.at[pl.ds(off, bw)], src_v)
            pltpu.sync_copy(dst_hbm.at[pl.ds(off, bw)], dst_v)
            pltpu.sync_copy(msk_hbm.at[pl.ds(off, bw)], msk_v)
            for j in range(bw // 16):
                sl = pl.ds(j * 16, 16)
                sv = src_v[sl]
                dv = dst_v[sl]
                x0 = (plsc.load_gather(xu0_v, [sv])
                      - plsc.load_gather(xu0_v, [dv]) + c0b)
                x1 = (plsc.load_gather(xu1_v, [sv])
                      - plsc.load_gather(xu1_v, [dv]) + c1b)
                m = jnp.maximum(x0, x1)
                e0 = jnp.exp(x0 - m)
                e1 = jnp.exp(x1 - m)
                rs = msk_v[sl] / (e0 + e1)
                q0_v[sl] = e0 * rs
                q1_v[sl] = e1 * rs
                i0_v[sl] = sv + cn0
                i1_v[sl] = sv + cn1
            pltpu.async_copy(xw_hbm.at[i0_v], r0_v, sem).wait()
            pltpu.async_copy(xw_hbm.at[i1_v], r1_v, sem).wait()

            def scale(g, _):
                g16 = pl.multiple_of(g * 16, 16)
                v0 = q0_v[pl.ds(g16, 16)]
                v1 = q1_v[pl.ds(g16, 16)]
                for e in range(16):
                    b0 = jnp.broadcast_to(v0[e], (16,))
                    b1 = jnp.broadcast_to(v1[e], (16,))
                    for jj in range(8):
                        sl2 = pl.ds(jj * 16, 16)
                        r0_v[g16 + e, sl2] = (r0_v[g16 + e, sl2] * b0
                                              + r1_v[g16 + e, sl2] * b1)
                return 0

            lax.fori_loop(0, bw // 16, scale, 0)
            pltpu.sync_copy(r0_v, acc.at[dst_v], add=True)
            return carry

        lax.fori_loop(0, nwin, window, 0)
        plsc.subcore_barrier()
        _rowsplit(s, lambda r0, nr: pltpu.sync_copy(
            acc.at[pl.ds(r0, nr)], out_hbm.at[c, pl.ds(r0, nr)]))

    return feast


@functools.partial(
    pl.kernel, out_type=jax.ShapeDtypeStruct((2, N, 128), _f32),
    mesh=_mesh(),
    compiler_params=pltpu.CompilerParams(needs_layout_passes=False),
    scratch_types=[
        pltpu.VMEM_SHARED((N, 128), _f32),
        pltpu.VMEM((128,), _i32),
        pltpu.VMEM((128,), _f32),
        pltpu.VMEM((128,), _f32),
        pltpu.VMEM((128, 128), _f32),
    ])
def _degcnt(dst_hbm, ew_hbm, msk_hbm, zero_hbm, out_hbm,
            dcacc, dst_v, ew_v, msk_v, d_v):
    c = lax.axis_index("c")
    s = lax.axis_index("s")
    _rowsplit(s, lambda r0, nr: pltpu.sync_copy(
        zero_hbm.at[pl.ds(r0, nr)], dcacc.at[pl.ds(r0, nr)]))
    plsc.subcore_barrier()
    base = s * CHUNK + c * (CHUNK // 2)
    zv = jnp.zeros((16,), _f32)

    def window(w, carry):
        off = base + w * 128
        pltpu.sync_copy(dst_hbm.at[pl.ds(off, 128)], dst_v)
        pltpu.sync_copy(ew_hbm.at[pl.ds(off, 128)], ew_v)
        pltpu.sync_copy(msk_hbm.at[pl.ds(off, 128)], msk_v)

        iot = lax.broadcasted_iota(_i32, (16,), 0)

        def build(g, _):
            g16 = pl.multiple_of(g * 16, 16)
            wv = ew_v[pl.ds(g16, 16)]
            mv = msk_v[pl.ds(g16, 16)]
            for e in range(16):
                bew = jnp.broadcast_to(wv[e], (16,))
                bm = jnp.broadcast_to(mv[e], (16,))
                d_v[g16 + e, pl.ds(0, 16)] = (
                    jnp.where(iot == 0, bew, 0.0)
                    + jnp.where(iot == 1, bm, 0.0))
                for jj in range(1, 8):
                    d_v[g16 + e, pl.ds(jj * 16, 16)] = zv
            return 0

        lax.fori_loop(0, 8, build, 0)
        pltpu.sync_copy(d_v, dcacc.at[dst_v], add=True)
        return carry

    lax.fori_loop(0, CHUNK // 128 // 2, window, 0)
    plsc.subcore_barrier()
    _rowsplit(s, lambda r0, nr: pltpu.sync_copy(
        dcacc.at[pl.ds(r0, nr)], out_hbm.at[c, pl.ds(r0, nr)]))


_spmm_cs = _make_spmm(True)
_spmm_es = _make_spmm(False)
_feast1 = _make_feast(True, 64)
_feast2 = _make_feast(False, 64)


def _dot(a, b):
    return jnp.dot(a, b, preferred_element_type=_f32)


def _tc_call(body, out_shapes, ins, in_specs, out_specs):
    return pl.pallas_call(
        body,
        grid=(N // ROW_BLK,),
        in_specs=in_specs,
        out_specs=out_specs,
        out_shape=out_shapes,
    )(*ins)


def _rows(k):
    return pl.BlockSpec((ROW_BLK, k), lambda i: (i, 0))


def _full(*shape):
    nd = len(shape)
    return pl.BlockSpec(shape, lambda i: (0,) * nd)


def _stk(nsplit, k):
    return pl.BlockSpec((nsplit, ROW_BLK, k), lambda i: (0, i, 0))


def _tc_feast_pre(x, W, upad):
    def body(x_ref, w_ref, u_ref, o1_ref, o2_ref):
        xb = x_ref[...]
        for j in range(4):
            o1_ref[j] = _dot(xb, w_ref[:, j * 128:(j + 1) * 128])
        o2_ref[...] = _dot(xb, u_ref[...])

    k = x.shape[1]
    return _tc_call(
        body,
        (jax.ShapeDtypeStruct((4, N, 128), _f32),
         jax.ShapeDtypeStruct((N, 128), _f32)),
        (x, W, upad),
        [_rows(k), _full(k, 512), _full(k, 128)],
        (_stk(4, 128), _rows(128)),
    )


def _tc_feast_mid(acc1, cinv2d, b1p, W2, u2pad):
    def body(a_ref, ci_ref, b_ref, w_ref, u_ref, o1_ref, o2_ref):
        ci = ci_ref[...]
        t0 = jnp.maximum(a_ref[0] * ci + b_ref[0:1, :128], 0.0)
        t1 = jnp.maximum(a_ref[1] * ci + b_ref[0:1, 128:], 0.0)
        h1 = jnp.concatenate([t0, t1], axis=1)
        for j in range(2):
            o1_ref[j] = _dot(h1, w_ref[:, j * 128:(j + 1) * 128])
        o2_ref[...] = _dot(h1, u_ref[...])

    return _tc_call(
        body,
        (jax.ShapeDtypeStruct((2, N, 128), _f32),
         jax.ShapeDtypeStruct((N, 128), _f32)),
        (acc1, cinv2d, b1p, W2, u2pad),
        [_stk(2, 128), _rows(128), _full(8, 256), _full(256, 256),
         _full(256, 128)],
        (_stk(2, 128), _rows(128)),
    )


def _tc_gcn_pre(acc2, cinv2d, b2p, Wl, blp, We1, dinv2d):
    def body(a_ref, ci_ref, b2_ref, wl_ref, bl_ref, we_ref, di_ref, o_ref):
        t = a_ref[0] + a_ref[1]
        t = jnp.maximum(t * ci_ref[...] + b2_ref[0:1], 0.0)
        h = _dot(t, wl_ref[...]) + bl_ref[0:1]
        di = di_ref[...]
        for j in range(2):
            o_ref[j] = di * _dot(h, we_ref[:, j * 128:(j + 1) * 128])

    return _tc_call(
        body,
        jax.ShapeDtypeStruct((2, N, 128), _f32),
        (acc2, cinv2d, b2p, Wl, blp, We1, dinv2d),
        [_stk(2, 128), _rows(128), _full(8, 128), _full(128, 128),
         _full(8, 128), _full(128, 256), _rows(128)],
        _stk(2, 128),
    )


def _tc_gcn_mid(agg, dinv2d, bp, Wn):
    def body(a_ref, di_ref, b_ref, w_ref, o_ref):
        di = di_ref[...]
        t0 = jnp.maximum(a_ref[0] * di + b_ref[0:1, :128], 0.0)
        t1 = jnp.maximum(a_ref[1] * di + b_ref[0:1, 128:], 0.0)
        t = jnp.concatenate([t0, t1], axis=1)
        for j in range(2):
            o_ref[j] = di * _dot(t, w_ref[:, j * 128:(j + 1) * 128])

    return _tc_call(
        body,
        jax.ShapeDtypeStruct((2, N, 128), _f32),
        (agg, dinv2d, bp, Wn),
        [_stk(2, 128), _rows(128), _full(8, 256), _full(256, 256)],
        _stk(2, 128),
    )


def _tc_gcn_last(agg, dinv2d, bp, Wn):
    def body(a_ref, di_ref, b_ref, w_ref, o_ref):
        di = di_ref[...]
        t0 = jnp.maximum(a_ref[0] * di + b_ref[0:1, :128], 0.0)
        t1 = jnp.maximum(a_ref[1] * di + b_ref[0:1, 128:], 0.0)
        t = jnp.concatenate([t0, t1], axis=1)
        o_ref[...] = di * _dot(t, w_ref[...])

    return _tc_call(
        body,
        jax.ShapeDtypeStruct((N, 128), _f32),
        (agg, dinv2d, bp, Wn),
        [_stk(2, 128), _rows(128), _full(8, 256), _full(256, 128)],
        _rows(128),
    )


def _tc_z(agg2, dinv2d, bmup, blvp, eps, Wd1):
    def body(a_ref, di_ref, bm_ref, bl_ref, e_ref, w_ref,
             mu_ref, lv_ref, o_ref):
        di = di_ref[...]
        mu = a_ref[0] * di + bm_ref[0:1]
        lv = a_ref[1] * di + bl_ref[0:1]
        mu_ref[...] = mu
        lv_ref[...] = lv
        z = mu + jnp.exp(0.5 * lv) * e_ref[...]
        for j in range(2):
            o_ref[j] = di * _dot(z, w_ref[:, j * 128:(j + 1) * 128])

    return _tc_call(
        body,
        (jax.ShapeDtypeStruct((N, 128), _f32),
         jax.ShapeDtypeStruct((N, 128), _f32),
         jax.ShapeDtypeStruct((2, N, 128), _f32)),
        (agg2, dinv2d, bmup, blvp, eps, Wd1),
        [_stk(2, 128), _rows(128), _full(8, 128), _full(8, 128),
         _rows(128), _full(128, 256)],
        (_rows(128), _rows(128), _stk(2, 128)),
    )


def _tc_final(agg4, dinv2d, bd2p):
    def body(a_ref, di_ref, b_ref, o_ref):
        t = a_ref[0] + a_ref[1]
        o_ref[...] = t * di_ref[...] + b_ref[0:1]

    return _tc_call(
        body,
        jax.ShapeDtypeStruct((N, 128), _f32),
        (agg4, dinv2d, bd2p),
        [_stk(2, 128), _rows(128), _full(8, 128)],
        _rows(128),
    )


def kernel(x, edge_index, edge_weight, W1, u1, c1, b1, W2, u2, c2, b2, Wl, bl,
           We1, be1, Wmu, bmu, Wlv, blv, Wd1, bd1, Wd2, bd2):
    idt = edge_index.dtype
    loop = jnp.arange(N, dtype=idt)
    padi = jnp.arange(NPAD, dtype=idt) % N
    src = jnp.concatenate([edge_index[0], loop, padi])
    dst = jnp.concatenate([edge_index[1], loop, padi])
    ew = jnp.concatenate([edge_weight, jnp.ones((N,), _f32),
                          jnp.zeros((NPAD,), _f32)])
    msk = jnp.concatenate([jnp.ones((E + N,), _f32),
                           jnp.zeros((NPAD,), _f32)])
    zeros128 = jnp.zeros((N, 128), _f32)

    def _padxu(v):
        return jnp.pad(v, (0, NXU - N))

    def pad_u(u):
        return jnp.zeros((u.shape[0], 128), _f32).at[:, :HEADS].set(u)

    def pad_b(b):
        return jnp.broadcast_to(b[None, :], (8, b.shape[0]))

    cpad = lambda cc: jnp.zeros((128,), _f32).at[:HEADS].set(cc)

    dc = _degcnt(dst, ew, msk, zeros128)
    deg = dc[0, :, 0] + dc[1, :, 0]
    cnt = dc[0, :, 1] + dc[1, :, 1]
    cinv2d = jnp.broadcast_to((1.0 / jnp.clip(cnt, 1.0, None))[:, None],
                              (N, 128))
    dinv = jnp.where(deg > 0, lax.rsqrt(deg), 0.0)
    dinv2d = jnp.broadcast_to(dinv[:, None], (N, 128))

    xw1stk, xu1p = _tc_feast_pre(x, W1, pad_u(u1))
    acc1 = _feast1(
        xw1stk.reshape(4 * N, 128), src, dst, msk,
        _padxu(xu1p[:, 0]), _padxu(xu1p[:, 1]), cpad(c1), zeros128)

    xw2stk, xu2p = _tc_feast_mid(acc1, cinv2d, pad_b(b1), W2, pad_u(u2))
    acc2 = _feast2(
        xw2stk.reshape(2 * N, 128), src, dst, msk,
        _padxu(xu2p[:, 0]), _padxu(xu2p[:, 1]), cpad(c2), zeros128)

    y1 = _tc_gcn_pre(acc2, cinv2d, pad_b(b2), Wl, pad_b(bl), We1, dinv2d)
    srcw = src.reshape(E2P // 128, 128)
    dstw = dst.reshape(E2P // 128, 128)
    eww = ew.reshape(E2P // 128, 128)
    agg1 = _spmm_cs(y1.reshape(2 * N, 128), srcw, dstw, eww, zeros128)
    Wml = jnp.concatenate([Wmu, Wlv], axis=1)
    y2 = _tc_gcn_mid(agg1, dinv2d, pad_b(be1), Wml)
    agg2 = _spmm_cs(y2.reshape(2 * N, 128), srcw, dstw, eww, zeros128)
    eps = jax.random.normal(jax.random.key(42), (N, LAT), dtype=_f32)
    mu, logvar, y3 = _tc_z(agg2, dinv2d, pad_b(bmu), pad_b(blv), eps, Wd1)
    agg3 = _spmm_cs(y3.reshape(2 * N, 128), srcw, dstw, eww, zeros128)
    y4 = _tc_gcn_last(agg3, dinv2d, pad_b(bd1), Wd2)
    agg4 = _spmm_es(y4, srcw, dstw, eww, zeros128)
    recon = _tc_final(agg4, dinv2d, bd2p=pad_b(bd2))
    return recon, mu, logvar
